# baseline (speedup 1.0000x reference)
import functools

import jax
import jax.numpy as jnp
from jax import lax
from jax.experimental import pallas as pl
from jax.experimental.pallas import tpu as pltpu
from jax.experimental.pallas import tpu_sc as plsc

K_NEIGHBORS = 16
EPS = 1e-5
BIG = 3.0e38


def _knn_body(x_ref, xt_ref, idx_ref, *, n, rows, k):
    b = pl.program_id(0)
    r = pl.program_id(1)
    xft = xt_ref[0]
    xrow = x_ref[0, pl.ds(r * rows, rows), :]

    dot = jax.lax.dot_general(
        xrow, xft, (((1,), (0,)), ((), ())),
        preferred_element_type=jnp.float32)
    sq_full = jnp.sum(xft * xft, axis=0, keepdims=True)
    sq_rows = jnp.sum(xrow * xrow, axis=1, keepdims=True)
    d = sq_rows - 2.0 * dot + sq_full

    colid = lax.broadcasted_iota(jnp.int32, (rows, n), 1)
    nbig = jnp.int32(n)
    vt = jnp.full((rows, 1), -BIG, jnp.float32)
    cols = []
    for _ in range(k):
        rm = jnp.min(jnp.where(d > vt, d, BIG), axis=1, keepdims=True)
        am = jnp.min(jnp.where(d == rm, colid, nbig), axis=1, keepdims=True)
        cols.append(am)
        vt = rm
    idx_ref[0] = jnp.concatenate(cols, axis=1) + b * n


def _knn_indices(x, xt, rows=256):
    bsz, n, c = x.shape
    grid = (bsz, n // rows)
    return pl.pallas_call(
        functools.partial(_knn_body, n=n, rows=rows, k=K_NEIGHBORS),
        grid=grid,
        in_specs=[
            pl.BlockSpec((1, n, c), lambda b, r: (b, 0, 0)),
            pl.BlockSpec((1, c, n), lambda b, r: (b, 0, 0)),
        ],
        out_specs=pl.BlockSpec((1, rows, K_NEIGHBORS), lambda b, r: (b, r, 0)),
        out_shape=jax.ShapeDtypeStruct((bsz, n, K_NEIGHBORS), jnp.int32),
    )(x, xt)


def _gather_max(x_flat, idx_flat):
    p_total, c = x_flat.shape
    k = K_NEIGHBORS
    info = plsc.get_sparse_core_info()
    nw = info.num_cores * info.num_subcores
    ppw = p_total // nw
    cp = 8
    n_chunks = ppw // cp
    lanes = info.num_lanes

    mesh = plsc.VectorSubcoreMesh(core_axis_name="c", subcore_axis_name="s")

    @functools.partial(
        pl.kernel, mesh=mesh,
        out_type=jax.ShapeDtypeStruct((p_total, c), jnp.float32),
        scratch_types=[
            pltpu.VMEM((ppw * k,), jnp.int32),
            pltpu.VMEM((cp * k, c), jnp.float32),
            pltpu.VMEM((cp * k, c), jnp.float32),
            pltpu.VMEM((cp, c), jnp.float32),
            pltpu.VMEM((cp, c), jnp.float32),
            pltpu.SemaphoreType.DMA,
            pltpu.SemaphoreType.DMA,
        ],
    )
    def gather_kernel(x_hbm, idx_hbm, out_hbm, idx_all,
                      rows_a, rows_b, out_a, out_b, sem_a, sem_b):
        wid = lax.axis_index("s") * info.num_cores + lax.axis_index("c")
        base = wid * ppw
        pltpu.sync_copy(idx_hbm.at[pl.ds(base * k, ppw * k)], idx_all)

        def start(g, rows, sem):
            pltpu.async_copy(
                x_hbm.at[idx_all.at[pl.ds(g * (cp * k), cp * k)]], rows, sem)

        def wait(rows, sem):
            pltpu.make_async_copy(x_hbm.at[pl.ds(0, cp * k)], rows, sem).wait()

        def compute(g, rows, out):
            def point(p, carry2):
                for ch in range(c // lanes):
                    acc = rows[p * k, pl.ds(ch * lanes, lanes)]
                    for j in range(1, k):
                        acc = jnp.maximum(
                            acc, rows[p * k + j, pl.ds(ch * lanes, lanes)])
                    out[p, pl.ds(ch * lanes, lanes)] = acc
                return carry2

            lax.fori_loop(0, cp, point, 0, unroll=False)
            pltpu.sync_copy(out, out_hbm.at[pl.ds(base + g * cp, cp)])

        n_pairs = n_chunks // 2
        start(0, rows_a, sem_a)

        def pair(h, carry):
            start(2 * h + 1, rows_b, sem_b)
            wait(rows_a, sem_a)
            compute(2 * h, rows_a, out_a)

            @pl.when(h + 1 < n_pairs)
            def _():
                start(2 * h + 2, rows_a, sem_a)

            wait(rows_b, sem_b)
            compute(2 * h + 1, rows_b, out_b)
            return carry

        lax.fori_loop(0, n_pairs, pair, 0, unroll=False)

    return gather_kernel(x_flat, idx_flat)


def _head_body(m_ref, w_ref, b_ref, g_ref, be_ref, out_ref):
    m = m_ref[...]
    w = w_ref[...]
    y = jax.lax.dot_general(
        m, w, (((1,), (1,)), ((), ())),
        preferred_element_type=jnp.float32) + b_ref[...]
    mu = jnp.mean(y, axis=0, keepdims=True)
    yc = y - mu
    var = jnp.mean(yc * yc, axis=0, keepdims=True)
    inv = g_ref[...] / jnp.sqrt(var + EPS)
    out_ref[...] = jnp.maximum(yc * inv + be_ref[...], 0.0)


def _head(m, w, b, gamma, beta):
    p_total, c = m.shape
    o = w.shape[0]
    return pl.pallas_call(
        _head_body,
        out_shape=jax.ShapeDtypeStruct((p_total, o), jnp.float32),
    )(m, w, b.reshape(1, o), gamma.reshape(1, o), beta.reshape(1, o))


def kernel(x, W, b, gamma, beta):
    bsz, n, c = x.shape
    xt = x.swapaxes(1, 2)
    idx = _knn_indices(x, xt)
    x_flat = x.reshape(bsz * n, c)
    idx_flat = idx.reshape(bsz * n * K_NEIGHBORS)
    m = _gather_max(x_flat, idx_flat)
    y = _head(m, W, b, gamma, beta)
    return y.reshape(bsz, n, -1)


# speedup vs baseline: 1.1494x; 1.1494x over previous
import functools

import jax
import jax.numpy as jnp
from jax import lax
from jax.experimental import pallas as pl
from jax.experimental.pallas import tpu as pltpu
from jax.experimental.pallas import tpu_sc as plsc

K_NEIGHBORS = 16
EPS = 1e-5
BIG = 3.0e38


def _knn_body(x_ref, xt_ref, idx_ref, *, n, rows, k):
    b = pl.program_id(0)
    r = pl.program_id(1)
    xft = xt_ref[0]
    xrow = x_ref[0, pl.ds(r * rows, rows), :]

    dot = jax.lax.dot_general(
        xrow, xft, (((1,), (0,)), ((), ())),
        preferred_element_type=jnp.float32)
    sq_full = jnp.sum(xft * xft, axis=0, keepdims=True)
    sq_rows = jnp.sum(xrow * xrow, axis=1, keepdims=True)
    d = sq_rows - 2.0 * dot + sq_full

    colf = lax.broadcasted_iota(jnp.int32, (rows, n), 1).astype(jnp.float32)
    vt = jnp.full((rows, 1), -BIG, jnp.float32)
    cols = []
    for _ in range(k):
        rm = jnp.min(jnp.where(d > vt, d, BIG), axis=1, keepdims=True)
        am = jnp.min(jnp.where(d == rm, colf, BIG), axis=1, keepdims=True)
        cols.append(am.astype(jnp.int32))
        vt = rm
    idx_ref[0] = jnp.concatenate(cols, axis=1) + b * n


def _knn_indices(x, xt, rows=256):
    bsz, n, c = x.shape
    grid = (bsz, n // rows)
    return pl.pallas_call(
        functools.partial(_knn_body, n=n, rows=rows, k=K_NEIGHBORS),
        grid=grid,
        in_specs=[
            pl.BlockSpec((1, n, c), lambda b, r: (b, 0, 0)),
            pl.BlockSpec((1, c, n), lambda b, r: (b, 0, 0)),
        ],
        out_specs=pl.BlockSpec((1, rows, K_NEIGHBORS), lambda b, r: (b, r, 0)),
        out_shape=jax.ShapeDtypeStruct((bsz, n, K_NEIGHBORS), jnp.int32),
    )(x, xt)


def _gather_max(x_flat, idx_flat):
    p_total, c = x_flat.shape
    k = K_NEIGHBORS
    info = plsc.get_sparse_core_info()
    nw = info.num_cores * info.num_subcores
    ppw = p_total // nw
    cp = 8
    n_chunks = ppw // cp
    lanes = info.num_lanes

    mesh = plsc.VectorSubcoreMesh(core_axis_name="c", subcore_axis_name="s")

    @functools.partial(
        pl.kernel, mesh=mesh,
        out_type=jax.ShapeDtypeStruct((p_total, c), jnp.float32),
        scratch_types=[
            pltpu.VMEM((ppw * k,), jnp.int32),
            pltpu.VMEM((cp * k, c), jnp.float32),
            pltpu.VMEM((cp * k, c), jnp.float32),
            pltpu.VMEM((cp, c), jnp.float32),
            pltpu.VMEM((cp, c), jnp.float32),
            pltpu.SemaphoreType.DMA,
            pltpu.SemaphoreType.DMA,
        ],
    )
    def gather_kernel(x_hbm, idx_hbm, out_hbm, idx_all,
                      rows_a, rows_b, out_a, out_b, sem_a, sem_b):
        wid = lax.axis_index("s") * info.num_cores + lax.axis_index("c")
        base = wid * ppw
        pltpu.sync_copy(idx_hbm.at[pl.ds(base * k, ppw * k)], idx_all)

        def start(g, rows, sem):
            pltpu.async_copy(
                x_hbm.at[idx_all.at[pl.ds(g * (cp * k), cp * k)]], rows, sem)

        def wait(rows, sem):
            pltpu.make_async_copy(x_hbm.at[pl.ds(0, cp * k)], rows, sem).wait()

        def compute(g, rows, out):
            def point(p, carry2):
                for ch in range(c // lanes):
                    acc = rows[p * k, pl.ds(ch * lanes, lanes)]
                    for j in range(1, k):
                        acc = jnp.maximum(
                            acc, rows[p * k + j, pl.ds(ch * lanes, lanes)])
                    out[p, pl.ds(ch * lanes, lanes)] = acc
                return carry2

            lax.fori_loop(0, cp, point, 0, unroll=False)
            pltpu.sync_copy(out, out_hbm.at[pl.ds(base + g * cp, cp)])

        n_pairs = n_chunks // 2
        start(0, rows_a, sem_a)

        def pair(h, carry):
            start(2 * h + 1, rows_b, sem_b)
            wait(rows_a, sem_a)
            compute(2 * h, rows_a, out_a)

            @pl.when(h + 1 < n_pairs)
            def _():
                start(2 * h + 2, rows_a, sem_a)

            wait(rows_b, sem_b)
            compute(2 * h + 1, rows_b, out_b)
            return carry

        lax.fori_loop(0, n_pairs, pair, 0, unroll=False)

    return gather_kernel(x_flat, idx_flat)


def _head_body(m_ref, w_ref, b_ref, g_ref, be_ref, out_ref):
    m = m_ref[...]
    w = w_ref[...]
    y = jax.lax.dot_general(
        m, w, (((1,), (1,)), ((), ())),
        preferred_element_type=jnp.float32) + b_ref[...]
    mu = jnp.mean(y, axis=0, keepdims=True)
    yc = y - mu
    var = jnp.mean(yc * yc, axis=0, keepdims=True)
    inv = g_ref[...] / jnp.sqrt(var + EPS)
    out_ref[...] = jnp.maximum(yc * inv + be_ref[...], 0.0)


def _head(m, w, b, gamma, beta):
    p_total, c = m.shape
    o = w.shape[0]
    return pl.pallas_call(
        _head_body,
        out_shape=jax.ShapeDtypeStruct((p_total, o), jnp.float32),
    )(m, w, b.reshape(1, o), gamma.reshape(1, o), beta.reshape(1, o))


def kernel(x, W, b, gamma, beta):
    bsz, n, c = x.shape
    xt = x.swapaxes(1, 2)
    idx = _knn_indices(x, xt)
    x_flat = x.reshape(bsz * n, c)
    idx_flat = idx.reshape(bsz * n * K_NEIGHBORS)
    m = _gather_max(x_flat, idx_flat)
    y = _head(m, W, b, gamma, beta)
    return y.reshape(bsz, n, -1)


# speedup vs baseline: 1.1616x; 1.0107x over previous
import functools

import jax
import jax.numpy as jnp
from jax import lax
from jax.experimental import pallas as pl
from jax.experimental.pallas import tpu as pltpu
from jax.experimental.pallas import tpu_sc as plsc

K_NEIGHBORS = 16
EPS = 1e-5
BIG = 3.0e38


def _knn_body(x_ref, xt_ref, idx_ref, *, n, rows, k):
    b = pl.program_id(0)
    r = pl.program_id(1)
    xft = xt_ref[0]
    xrow = x_ref[0, pl.ds(r * rows, rows), :]

    dot = jax.lax.dot_general(
        xrow, xft, (((1,), (0,)), ((), ())),
        preferred_element_type=jnp.float32)
    sq_full = jnp.sum(xft * xft, axis=0, keepdims=True)
    sq_rows = jnp.sum(xrow * xrow, axis=1, keepdims=True)
    d = sq_rows - 2.0 * dot + sq_full

    colf = lax.broadcasted_iota(jnp.int32, (rows, n), 1).astype(jnp.float32)
    cols = []
    for _ in range(k):
        rm = jnp.min(d, axis=1, keepdims=True)
        am = jnp.min(jnp.where(d == rm, colf, BIG), axis=1, keepdims=True)
        cols.append(am.astype(jnp.int32))
        d = jnp.where(colf == am, BIG, d)
    idx_ref[0] = jnp.concatenate(cols, axis=1) + b * n


def _knn_indices(x, xt, rows=256):
    bsz, n, c = x.shape
    grid = (bsz, n // rows)
    return pl.pallas_call(
        functools.partial(_knn_body, n=n, rows=rows, k=K_NEIGHBORS),
        grid=grid,
        in_specs=[
            pl.BlockSpec((1, n, c), lambda b, r: (b, 0, 0)),
            pl.BlockSpec((1, c, n), lambda b, r: (b, 0, 0)),
        ],
        out_specs=pl.BlockSpec((1, rows, K_NEIGHBORS), lambda b, r: (b, r, 0)),
        out_shape=jax.ShapeDtypeStruct((bsz, n, K_NEIGHBORS), jnp.int32),
    )(x, xt)


def _gather_max(x_flat, idx_flat):
    p_total, c = x_flat.shape
    k = K_NEIGHBORS
    info = plsc.get_sparse_core_info()
    nw = info.num_cores * info.num_subcores
    ppw = p_total // nw
    cp = 8
    n_chunks = ppw // cp
    lanes = info.num_lanes

    mesh = plsc.VectorSubcoreMesh(core_axis_name="c", subcore_axis_name="s")

    @functools.partial(
        pl.kernel, mesh=mesh,
        out_type=jax.ShapeDtypeStruct((p_total, c), jnp.float32),
        scratch_types=[
            pltpu.VMEM((ppw * k,), jnp.int32),
            pltpu.VMEM((cp * k, c), jnp.float32),
            pltpu.VMEM((cp * k, c), jnp.float32),
            pltpu.VMEM((cp, c), jnp.float32),
            pltpu.VMEM((cp, c), jnp.float32),
            pltpu.SemaphoreType.DMA,
            pltpu.SemaphoreType.DMA,
        ],
    )
    def gather_kernel(x_hbm, idx_hbm, out_hbm, idx_all,
                      rows_a, rows_b, out_a, out_b, sem_a, sem_b):
        wid = lax.axis_index("s") * info.num_cores + lax.axis_index("c")
        base = wid * ppw
        pltpu.sync_copy(idx_hbm.at[pl.ds(base * k, ppw * k)], idx_all)

        def start(g, rows, sem):
            pltpu.async_copy(
                x_hbm.at[idx_all.at[pl.ds(g * (cp * k), cp * k)]], rows, sem)

        def wait(rows, sem):
            pltpu.make_async_copy(x_hbm.at[pl.ds(0, cp * k)], rows, sem).wait()

        def compute(g, rows, out):
            def point(p, carry2):
                for ch in range(c // lanes):
                    acc = rows[p * k, pl.ds(ch * lanes, lanes)]
                    for j in range(1, k):
                        acc = jnp.maximum(
                            acc, rows[p * k + j, pl.ds(ch * lanes, lanes)])
                    out[p, pl.ds(ch * lanes, lanes)] = acc
                return carry2

            lax.fori_loop(0, cp, point, 0, unroll=False)
            pltpu.sync_copy(out, out_hbm.at[pl.ds(base + g * cp, cp)])

        n_pairs = n_chunks // 2
        start(0, rows_a, sem_a)

        def pair(h, carry):
            start(2 * h + 1, rows_b, sem_b)
            wait(rows_a, sem_a)
            compute(2 * h, rows_a, out_a)

            @pl.when(h + 1 < n_pairs)
            def _():
                start(2 * h + 2, rows_a, sem_a)

            wait(rows_b, sem_b)
            compute(2 * h + 1, rows_b, out_b)
            return carry

        lax.fori_loop(0, n_pairs, pair, 0, unroll=False)

    return gather_kernel(x_flat, idx_flat)


def _head_body(m_ref, w_ref, b_ref, g_ref, be_ref, out_ref):
    m = m_ref[...]
    w = w_ref[...]
    y = jax.lax.dot_general(
        m, w, (((1,), (1,)), ((), ())),
        preferred_element_type=jnp.float32) + b_ref[...]
    mu = jnp.mean(y, axis=0, keepdims=True)
    yc = y - mu
    var = jnp.mean(yc * yc, axis=0, keepdims=True)
    inv = g_ref[...] / jnp.sqrt(var + EPS)
    out_ref[...] = jnp.maximum(yc * inv + be_ref[...], 0.0)


def _head(m, w, b, gamma, beta):
    p_total, c = m.shape
    o = w.shape[0]
    return pl.pallas_call(
        _head_body,
        out_shape=jax.ShapeDtypeStruct((p_total, o), jnp.float32),
    )(m, w, b.reshape(1, o), gamma.reshape(1, o), beta.reshape(1, o))


def kernel(x, W, b, gamma, beta):
    bsz, n, c = x.shape
    xt = x.swapaxes(1, 2)
    idx = _knn_indices(x, xt)
    x_flat = x.reshape(bsz * n, c)
    idx_flat = idx.reshape(bsz * n * K_NEIGHBORS)
    m = _gather_max(x_flat, idx_flat)
    y = _head(m, W, b, gamma, beta)
    return y.reshape(bsz, n, -1)
